# baseline (speedup 1.0000x reference)
import functools

import jax
import jax.numpy as jnp
from jax.experimental import pallas as pl
from jax.experimental.pallas import tpu as pltpu

ALPHA = 0.7
BETA = 0.3
EPS = 1e-12
BIG = 1e30


def _knn_min_kernel(qt_ref, nb_ref, pb_ref, out_ref,
                    mneg_ref, mpos_ref, nb2_ref, pb2_ref):
    j = pl.program_id(0)
    i = pl.program_id(1)
    nbj = pl.num_programs(0)
    qb = qt_ref.shape[1]

    qt = qt_ref[:]
    nb = nb_ref[:]
    pb = pb_ref[:]

    @pl.when(i == 0)
    def _norms():
        nb2_ref[:] = jnp.sum(nb * nb, axis=1, keepdims=True)
        pb2_ref[:] = jnp.sum(pb * pb, axis=1, keepdims=True)

    dn = jax.lax.dot_general(nb, qt, (((1,), (0,)), ((), ())),
                             preferred_element_type=jnp.float32)
    dp = jax.lax.dot_general(pb, qt, (((1,), (0,)), ((), ())),
                             preferred_element_type=jnp.float32)

    mn = jnp.min(nb2_ref[:] + dn, axis=0, keepdims=True)
    mp = jnp.min(pb2_ref[:] + dp, axis=0, keepdims=True)

    sl = pl.ds(i * qb, qb)
    prev_n = jnp.where(j == 0, BIG, mneg_ref[:, sl])
    prev_p = jnp.where(j == 0, BIG, mpos_ref[:, sl])
    acc_n = jnp.minimum(prev_n, mn)
    acc_p = jnp.minimum(prev_p, mp)
    mneg_ref[:, sl] = acc_n
    mpos_ref[:, sl] = acc_p

    @pl.when(j == nbj - 1)
    def _fin():
        q2 = 0.25 * jnp.sum(qt * qt, axis=0, keepdims=True)
        dneg = jnp.sqrt(jnp.maximum(q2 + acc_n, 0.0) + EPS)
        dpos = jnp.sqrt(jnp.maximum(q2 + acc_p, 0.0) + EPS)
        out_ref[:] = ALPHA * dneg - BETA * dpos


@functools.partial(jax.jit, static_argnames=("qb", "bb"))
def _run(queries, neg_bank, pos_bank, qb, bb):
    nq, d = queries.shape
    n = neg_bank.shape[0]
    qt = -2.0 * queries.T
    grid = (n // bb, nq // qb)
    out = pl.pallas_call(
        _knn_min_kernel,
        grid=grid,
        in_specs=[
            pl.BlockSpec((d, qb), lambda j, i: (0, i)),
            pl.BlockSpec((bb, d), lambda j, i: (j, 0)),
            pl.BlockSpec((bb, d), lambda j, i: (j, 0)),
        ],
        out_specs=pl.BlockSpec((1, qb), lambda j, i: (0, i)),
        out_shape=jax.ShapeDtypeStruct((1, nq), jnp.float32),
        scratch_shapes=[
            pltpu.VMEM((1, nq), jnp.float32),
            pltpu.VMEM((1, nq), jnp.float32),
            pltpu.VMEM((bb, 1), jnp.float32),
            pltpu.VMEM((bb, 1), jnp.float32),
        ],
    )(qt, neg_bank, pos_bank)
    return out[0]


def kernel(queries, neg_bank, pos_bank):
    return _run(queries, neg_bank, pos_bank, qb=896, bb=1000)


# speedup vs baseline: 1.0920x; 1.0920x over previous
import functools

import jax
import jax.numpy as jnp
from jax.experimental import pallas as pl
from jax.experimental.pallas import tpu as pltpu

ALPHA = 0.7
BETA = 0.3
EPS = 1e-12
BIG = 1e30


def _knn_min_kernel(qt_ref, nb_ref, pb_ref, out_ref, mneg_ref, mpos_ref):
    j = pl.program_id(0)
    i = pl.program_id(1)
    nbj = pl.num_programs(0)
    qb = qt_ref.shape[1]

    qt = qt_ref[:]
    nb = nb_ref[:]
    pb = pb_ref[:]

    dn = jax.lax.dot_general(nb, qt, (((1,), (0,)), ((), ())),
                             preferred_element_type=jnp.float32)
    dp = jax.lax.dot_general(pb, qt, (((1,), (0,)), ((), ())),
                             preferred_element_type=jnp.float32)
    nb2 = jnp.sum(nb * nb, axis=1, keepdims=True)
    pb2 = jnp.sum(pb * pb, axis=1, keepdims=True)

    mn = jnp.min(nb2 + dn, axis=0, keepdims=True)
    mp = jnp.min(pb2 + dp, axis=0, keepdims=True)

    sl = pl.ds(i * qb, qb)
    prev_n = jnp.where(j == 0, BIG, mneg_ref[:, sl])
    prev_p = jnp.where(j == 0, BIG, mpos_ref[:, sl])
    acc_n = jnp.minimum(prev_n, mn)
    acc_p = jnp.minimum(prev_p, mp)
    mneg_ref[:, sl] = acc_n
    mpos_ref[:, sl] = acc_p

    @pl.when(j == nbj - 1)
    def _fin():
        q2 = 0.25 * jnp.sum(qt * qt, axis=0, keepdims=True)
        dneg = jnp.sqrt(jnp.maximum(q2 + acc_n, 0.0) + EPS)
        dpos = jnp.sqrt(jnp.maximum(q2 + acc_p, 0.0) + EPS)
        out_ref[:] = ALPHA * dneg - BETA * dpos


@functools.partial(jax.jit, static_argnames=("qb", "bb"))
def _run(queries, neg_bank, pos_bank, qb, bb):
    nq, d = queries.shape
    n = neg_bank.shape[0]
    qt = -2.0 * queries.T
    grid = (n // bb, nq // qb)
    out = pl.pallas_call(
        _knn_min_kernel,
        grid=grid,
        in_specs=[
            pl.BlockSpec((d, qb), lambda j, i: (0, i)),
            pl.BlockSpec((bb, d), lambda j, i: (j, 0)),
            pl.BlockSpec((bb, d), lambda j, i: (j, 0)),
        ],
        out_specs=pl.BlockSpec((1, qb), lambda j, i: (0, i)),
        out_shape=jax.ShapeDtypeStruct((1, nq), jnp.float32),
        scratch_shapes=[
            pltpu.VMEM((1, nq), jnp.float32),
            pltpu.VMEM((1, nq), jnp.float32),
        ],
    )(qt, neg_bank, pos_bank)
    return out[0]


def kernel(queries, neg_bank, pos_bank):
    return _run(queries, neg_bank, pos_bank, qb=896, bb=1000)


# speedup vs baseline: 1.2141x; 1.1118x over previous
import functools

import jax
import jax.numpy as jnp
from jax.experimental import pallas as pl
from jax.experimental.pallas import tpu as pltpu

ALPHA = 0.7
BETA = 0.3
EPS = 1e-12
BIG = 1e30


def _knn_min_kernel(qt_ref, nb_ref, pb_ref, out_ref, mneg_ref, mpos_ref):
    j = pl.program_id(0)
    i = pl.program_id(1)
    nbj = pl.num_programs(0)
    qb = qt_ref.shape[1]

    qt = qt_ref[:]
    nb = nb_ref[:]
    pb = pb_ref[:]

    nb2 = jnp.sum(nb * nb, axis=1, keepdims=True)
    dn = jax.lax.dot_general(nb, qt, (((1,), (0,)), ((), ())),
                             preferred_element_type=jnp.float32)
    mn = jnp.min(nb2 - 2.0 * dn, axis=0, keepdims=True)

    pb2 = jnp.sum(pb * pb, axis=1, keepdims=True)
    dp = jax.lax.dot_general(pb, qt, (((1,), (0,)), ((), ())),
                             preferred_element_type=jnp.float32)
    mp = jnp.min(pb2 - 2.0 * dp, axis=0, keepdims=True)

    sl = pl.ds(i * qb, qb)
    prev_n = jnp.where(j == 0, BIG, mneg_ref[:, sl])
    prev_p = jnp.where(j == 0, BIG, mpos_ref[:, sl])
    acc_n = jnp.minimum(prev_n, mn)
    acc_p = jnp.minimum(prev_p, mp)
    mneg_ref[:, sl] = acc_n
    mpos_ref[:, sl] = acc_p

    @pl.when(j == nbj - 1)
    def _fin():
        q2 = jnp.sum(qt * qt, axis=0, keepdims=True)
        dneg = jnp.sqrt(jnp.maximum(q2 + acc_n, 0.0) + EPS)
        dpos = jnp.sqrt(jnp.maximum(q2 + acc_p, 0.0) + EPS)
        out_ref[:] = ALPHA * dneg - BETA * dpos


@functools.partial(jax.jit, static_argnames=("qb", "bb"))
def _run(queries, neg_bank, pos_bank, qb, bb):
    nq, d = queries.shape
    n = neg_bank.shape[0]
    nq_pad = ((nq + qb - 1) // qb) * qb
    qt = jnp.pad(queries.T, ((0, 0), (0, nq_pad - nq)))
    grid = (n // bb, nq_pad // qb)
    out = pl.pallas_call(
        _knn_min_kernel,
        grid=grid,
        in_specs=[
            pl.BlockSpec((d, qb), lambda j, i: (0, i)),
            pl.BlockSpec((bb, d), lambda j, i: (j, 0)),
            pl.BlockSpec((bb, d), lambda j, i: (j, 0)),
        ],
        out_specs=pl.BlockSpec((1, qb), lambda j, i: (0, i)),
        out_shape=jax.ShapeDtypeStruct((1, nq_pad), jnp.float32),
        scratch_shapes=[
            pltpu.VMEM((1, nq_pad), jnp.float32),
            pltpu.VMEM((1, nq_pad), jnp.float32),
        ],
    )(qt, neg_bank, pos_bank)
    return out[0, :nq]


def kernel(queries, neg_bank, pos_bank):
    return _run(queries, neg_bank, pos_bank, qb=1280, bb=1000)
